# Initial kernel scaffold; baseline (speedup 1.0000x reference)
import functools

import jax
import jax.numpy as jnp
from jax import lax
from jax.experimental import pallas as pl
from jax.experimental.pallas import tpu as pltpu
from jax.experimental.pallas import tpu_sc as plsc

_NUM_CORES = 2
_NUM_SUBCORES = 16
_NW = _NUM_CORES * _NUM_SUBCORES

_D = 1000
_CHUNK = 64


@functools.partial(jax.jit, static_argnames=("n_tokens",))
def _sc_embedding_gather(idx_flat, table, *, n_tokens):
    b_per_w = n_tokens // _NW
    n_chunks = b_per_w // _CHUNK
    idx3 = idx_flat.reshape(_NW, n_chunks, _CHUNK).astype(jnp.int32)

    mesh = plsc.VectorSubcoreMesh(
        core_axis_name="c",
        subcore_axis_name="s",
        num_cores=_NUM_CORES,
        num_subcores=_NUM_SUBCORES,
    )

    @functools.partial(
        pl.kernel,
        out_type=jax.ShapeDtypeStruct((n_tokens, _D), jnp.float32),
        mesh=mesh,
        scratch_types=[
            pltpu.VMEM((n_chunks, _CHUNK), jnp.int32),
            pltpu.VMEM((_CHUNK, _D), jnp.float32),
            pltpu.SemaphoreType.DMA,
        ],
    )
    def gather_kernel(table_hbm, idx_hbm, out_hbm, idx_v, buf, gsem):
        wid = lax.axis_index("s") * _NUM_CORES + lax.axis_index("c")
        base = wid * b_per_w
        pltpu.sync_copy(idx_hbm.at[wid], idx_v)

        @pl.loop(0, n_chunks)
        def _(g):
            pltpu.async_copy(table_hbm.at[idx_v.at[g]], buf, gsem).wait()
            pltpu.sync_copy(buf, out_hbm.at[pl.ds(base + g * _CHUNK, _CHUNK)])

    return gather_kernel(table, idx3)


def kernel(idx, token_embedding_table):
    B, T = idx.shape
    n_tokens = B * T
    out = _sc_embedding_gather(
        idx.reshape(n_tokens), token_embedding_table, n_tokens=n_tokens
    )
    return out.reshape(B, T, _D)


# speedup vs baseline: 1.4179x; 1.4179x over previous
import functools

import jax
import jax.numpy as jnp
from jax import lax
from jax.experimental import pallas as pl
from jax.experimental.pallas import tpu as pltpu
from jax.experimental.pallas import tpu_sc as plsc

_NUM_CORES = 2
_NUM_SUBCORES = 16
_NW = _NUM_CORES * _NUM_SUBCORES

_D = 1000
_DP = 1024
_CHUNK = 32


@functools.partial(jax.jit, static_argnames=("n_tokens",))
def _sc_embedding_gather(idx_flat, table, *, n_tokens):
    b_per_w = n_tokens // _NW
    n_chunks = b_per_w // _CHUNK
    idx3 = idx_flat.reshape(_NW, n_chunks, _CHUNK).astype(jnp.int32)
    table_p = jnp.pad(table, ((0, 0), (0, _DP - _D)))

    mesh = plsc.VectorSubcoreMesh(
        core_axis_name="c",
        subcore_axis_name="s",
        num_cores=_NUM_CORES,
        num_subcores=_NUM_SUBCORES,
    )

    @functools.partial(
        pl.kernel,
        out_type=jax.ShapeDtypeStruct((n_tokens, _DP), jnp.float32),
        mesh=mesh,
        scratch_types=[
            pltpu.VMEM((n_chunks, _CHUNK), jnp.int32),
            pltpu.VMEM((2, _CHUNK, _DP), jnp.float32),
            pltpu.SemaphoreType.DMA,
            pltpu.SemaphoreType.DMA,
        ],
    )
    def gather_kernel(table_hbm, idx_hbm, out_hbm, idx_v, buf, gsem, ssem):
        wid = lax.axis_index("s") * _NUM_CORES + lax.axis_index("c")
        base = wid * b_per_w
        pltpu.sync_copy(idx_hbm.at[wid], idx_v)

        pltpu.make_async_copy(table_hbm.at[idx_v.at[0]], buf.at[0], gsem).start()

        @pl.loop(0, n_chunks)
        def _(g):
            slot = lax.rem(g, 2)
            nslot = lax.rem(g + 1, 2)

            @pl.when(g + 1 < n_chunks)
            def _():
                @pl.when(g >= 1)
                def _():
                    pltpu.make_async_copy(
                        buf.at[nslot],
                        out_hbm.at[pl.ds(0, _CHUNK)],
                        ssem,
                    ).wait()

                pltpu.make_async_copy(
                    table_hbm.at[idx_v.at[g + 1]], buf.at[nslot], gsem
                ).start()

            pltpu.make_async_copy(
                table_hbm.at[idx_v.at[g]], buf.at[slot], gsem
            ).wait()
            pltpu.make_async_copy(
                buf.at[slot],
                out_hbm.at[pl.ds(base + g * _CHUNK, _CHUNK)],
                ssem,
            ).start()

        pltpu.make_async_copy(
            buf.at[0], out_hbm.at[pl.ds(0, 2 * _CHUNK)], ssem
        ).wait()

    return gather_kernel(table_p, idx3)


def kernel(idx, token_embedding_table):
    B, T = idx.shape
    n_tokens = B * T
    out = _sc_embedding_gather(
        idx.reshape(n_tokens), token_embedding_table, n_tokens=n_tokens
    )
    return out[:, :_D].reshape(B, T, _D)
